# baseline (speedup 1.0000x reference)
import jax
import jax.numpy as jnp
from jax import lax
from jax.experimental import pallas as pl
from jax.experimental.pallas import tpu as pltpu

B, L, DIN, DOUT, E = 4, 2048, 64, 128, 8
C = 2 * DOUT
K = 5
PAD = K // 2
CH = 512
NCH = L // CH
DA = DIN + 1
BB = 4


def _moe_chunk(xa, gw, ew):
    logits = jnp.dot(xa, gw, preferred_element_type=jnp.float32)
    m = jnp.max(logits, axis=-1, keepdims=True)
    p = jnp.exp(logits - m)
    gates = (p / jnp.sum(p, axis=-1, keepdims=True)).astype(jnp.bfloat16)
    u = jnp.dot(xa, ew, preferred_element_type=jnp.float32).astype(jnp.bfloat16)
    q = u + u * jnp.tanh(u)
    acc = gates[:, 0:1] * q[:, 0:DOUT]
    for e in range(1, E):
        acc += gates[:, e:e + 1] * q[:, e * DOUT:(e + 1) * DOUT]
    return acc


def _body(a_ref, d_ref, gwt_ref, gbt_ref, ewt_ref, ebt_ref,
          gwd_ref, gbd_ref, ewd_ref, ebd_ref, wk_ref, cb_ref,
          out_ref, gwt_s, ewt_s, gwd_s, ewd_s):
    @pl.when(pl.program_id(0) == 0)
    def _init():
        for gs, gref, gbref, es, eref, ebref in (
                (gwt_s, gwt_ref, gbt_ref, ewt_s, ewt_ref, ebt_ref),
                (gwd_s, gwd_ref, gbd_ref, ewd_s, ewd_ref, ebd_ref)):
            gs[0:DIN, :] = gref[...].astype(jnp.bfloat16)
            gs[DIN:DA, :] = gbref[...].astype(jnp.bfloat16)
            for e in range(E):
                sl = slice(e * DOUT, (e + 1) * DOUT)
                es[0:DIN, sl] = (0.5 * eref[e]).astype(jnp.bfloat16)
                es[DIN:DA, sl] = (0.5 * ebref[e:e + 1, :]).astype(jnp.bfloat16)

    gwt, ewt = gwt_s[...], ewt_s[...]
    gwd, ewd = gwd_s[...], ewd_s[...]
    cb = cb_ref[...]
    zpad = jnp.zeros((PAD, C), jnp.bfloat16)

    for bb in range(BB):
        for c in range(NCH):
            lo = max(0, c * CH - PAD)
            hi = min(L, (c + 1) * CH + PAD)
            n = hi - lo
            ones = jnp.ones((n, 1), jnp.bfloat16)
            xa = jnp.concatenate(
                [a_ref[bb, pl.ds(lo, n), :].astype(jnp.bfloat16), ones], axis=1)
            xd = jnp.concatenate(
                [d_ref[bb, pl.ds(lo, n), :].astype(jnp.bfloat16), ones], axis=1)
            fa = _moe_chunk(xa, gwt, ewt)
            fd = _moe_chunk(xd, gwd, ewd)
            comb = jnp.concatenate([fa, fd], axis=1)
            if lo == 0:
                comb = jnp.concatenate([zpad, comb], axis=0)
            if hi == L:
                comb = jnp.concatenate([comb, zpad], axis=0)
            y = jnp.dot(lax.slice(comb, (0, 0), (CH, C)), wk_ref[0],
                        preferred_element_type=jnp.float32)
            for k in range(1, K):
                y += jnp.dot(lax.slice(comb, (k, 0), (k + CH, C)), wk_ref[k],
                             preferred_element_type=jnp.float32)
            out_ref[bb, pl.ds(c * CH, CH), :] = jnp.maximum(y + cb, 0.0)


@jax.jit
def kernel(a, d, gate_Wt, gate_bt, exp_Wt, exp_bt,
           gate_Wd, gate_bd, exp_Wd, exp_bd, conv_W, conv_b):
    wk = jnp.transpose(conv_W, (2, 1, 0)).astype(jnp.bfloat16)
    gbt = gate_bt.reshape(1, E)
    gbd = gate_bd.reshape(1, E)
    cb = conv_b.reshape(1, C)

    full = lambda shape: pl.BlockSpec(shape, lambda b: (0,) * len(shape))
    return pl.pallas_call(
        _body,
        grid=(B // BB,),
        in_specs=[
            pl.BlockSpec((BB, L, DIN), lambda b: (b, 0, 0)),
            pl.BlockSpec((BB, L, DIN), lambda b: (b, 0, 0)),
            full((DIN, E)), full((1, E)), full((E, DIN, DOUT)), full((E, DOUT)),
            full((DIN, E)), full((1, E)), full((E, DIN, DOUT)), full((E, DOUT)),
            full((K, C, C)), full((1, C)),
        ],
        out_specs=pl.BlockSpec((BB, L, C), lambda b: (b, 0, 0)),
        out_shape=jax.ShapeDtypeStruct((B, L, C), jnp.float32),
        compiler_params=pltpu.CompilerParams(dimension_semantics=("arbitrary",)),
        scratch_shapes=[
            pltpu.VMEM((DA, E), jnp.bfloat16),
            pltpu.VMEM((DA, E * DOUT), jnp.bfloat16),
            pltpu.VMEM((DA, E), jnp.bfloat16),
            pltpu.VMEM((DA, E * DOUT), jnp.bfloat16),
        ],
    )(a, d, gate_Wt, gbt, exp_Wt, exp_bt, gate_Wd, gbd, exp_Wd, exp_bd, wk, cb)


# speedup vs baseline: 1.0421x; 1.0421x over previous
import jax
import jax.numpy as jnp
from jax import lax
from jax.experimental import pallas as pl
from jax.experimental.pallas import tpu as pltpu

B, L, DIN, DOUT, E = 4, 2048, 64, 128, 8
C = 2 * DOUT
K = 5
PAD = K // 2
CH = 512
NCH = L // CH
DA = DIN + 1
BB = 2


def _moe_chunk(xa, gw, ew):
    logits = jnp.dot(xa, gw, preferred_element_type=jnp.float32)
    m = jnp.max(logits, axis=-1, keepdims=True)
    p = jnp.exp(logits - m)
    gates = (p / jnp.sum(p, axis=-1, keepdims=True)).astype(jnp.bfloat16)
    u = jnp.dot(xa, ew, preferred_element_type=jnp.float32).astype(jnp.bfloat16)
    q = u + u * jnp.tanh(u)
    acc = gates[:, 0:1] * q[:, 0:DOUT]
    for e in range(1, E):
        acc += gates[:, e:e + 1] * q[:, e * DOUT:(e + 1) * DOUT]
    return acc


def _body(a_ref, d_ref, gwt_ref, gbt_ref, ewt_ref, ebt_ref,
          gwd_ref, gbd_ref, ewd_ref, ebd_ref, wk_ref, cb_ref,
          out_ref, gwt_s, ewt_s, gwd_s, ewd_s):
    @pl.when(pl.program_id(0) == 0)
    def _init():
        for gs, gref, gbref, es, eref, ebref in (
                (gwt_s, gwt_ref, gbt_ref, ewt_s, ewt_ref, ebt_ref),
                (gwd_s, gwd_ref, gbd_ref, ewd_s, ewd_ref, ebd_ref)):
            gs[0:DIN, :] = gref[...].astype(jnp.bfloat16)
            gs[DIN:DA, :] = gbref[...].astype(jnp.bfloat16)
            for e in range(E):
                sl = slice(e * DOUT, (e + 1) * DOUT)
                es[0:DIN, sl] = (0.5 * eref[e]).astype(jnp.bfloat16)
                es[DIN:DA, sl] = (0.5 * ebref[e:e + 1, :]).astype(jnp.bfloat16)

    gwt, ewt = gwt_s[...], ewt_s[...]
    gwd, ewd = gwd_s[...], ewd_s[...]
    cb = cb_ref[...]
    zpad = jnp.zeros((PAD, C), jnp.bfloat16)

    for bb in range(BB):
        for c in range(NCH):
            lo = max(0, c * CH - PAD)
            hi = min(L, (c + 1) * CH + PAD)
            n = hi - lo
            ones = jnp.ones((n, 1), jnp.bfloat16)
            xa = jnp.concatenate([a_ref[bb, pl.ds(lo, n), :], ones], axis=1)
            xd = jnp.concatenate([d_ref[bb, pl.ds(lo, n), :], ones], axis=1)
            fa = _moe_chunk(xa, gwt, ewt)
            fd = _moe_chunk(xd, gwd, ewd)
            comb = jnp.concatenate([fa, fd], axis=1)
            if lo == 0:
                comb = jnp.concatenate([zpad, comb], axis=0)
            if hi == L:
                comb = jnp.concatenate([comb, zpad], axis=0)
            y = jnp.dot(lax.slice(comb, (0, 0), (CH, C)), wk_ref[0],
                        preferred_element_type=jnp.float32)
            for k in range(1, K):
                y += jnp.dot(lax.slice(comb, (k, 0), (k + CH, C)), wk_ref[k],
                             preferred_element_type=jnp.float32)
            out_ref[bb, pl.ds(c * CH, CH), :] = jnp.maximum(y + cb, 0.0)


@jax.jit
def kernel(a, d, gate_Wt, gate_bt, exp_Wt, exp_bt,
           gate_Wd, gate_bd, exp_Wd, exp_bd, conv_W, conv_b):
    a16 = a.astype(jnp.bfloat16)
    d16 = d.astype(jnp.bfloat16)
    wk = jnp.transpose(conv_W, (2, 1, 0)).astype(jnp.bfloat16)
    gbt = gate_bt.reshape(1, E)
    gbd = gate_bd.reshape(1, E)
    cb = conv_b.reshape(1, C)

    full = lambda shape: pl.BlockSpec(shape, lambda b: (0,) * len(shape))
    return pl.pallas_call(
        _body,
        grid=(B // BB,),
        in_specs=[
            pl.BlockSpec((BB, L, DIN), lambda b: (b, 0, 0)),
            pl.BlockSpec((BB, L, DIN), lambda b: (b, 0, 0)),
            full((DIN, E)), full((1, E)), full((E, DIN, DOUT)), full((E, DOUT)),
            full((DIN, E)), full((1, E)), full((E, DIN, DOUT)), full((E, DOUT)),
            full((K, C, C)), full((1, C)),
        ],
        out_specs=pl.BlockSpec((BB, L, C), lambda b: (b, 0, 0)),
        out_shape=jax.ShapeDtypeStruct((B, L, C), jnp.float32),
        compiler_params=pltpu.CompilerParams(dimension_semantics=("arbitrary",)),
        scratch_shapes=[
            pltpu.VMEM((DA, E), jnp.bfloat16),
            pltpu.VMEM((DA, E * DOUT), jnp.bfloat16),
            pltpu.VMEM((DA, E), jnp.bfloat16),
            pltpu.VMEM((DA, E * DOUT), jnp.bfloat16),
        ],
    )(a16, d16, gate_Wt, gbt, exp_Wt, exp_bt, gate_Wd, gbd, exp_Wd, exp_bd, wk, cb)
